# Initial kernel scaffold; baseline (speedup 1.0000x reference)
import jax
import jax.numpy as jnp
from jax.experimental import pallas as pl


def kernel(x, table):
    raise NotImplementedError("write your pallas kernel here")


# speedup vs baseline: 1.6960x; 1.6960x over previous
import functools

import jax
import jax.numpy as jnp
from jax import lax
from jax.experimental import pallas as pl
from jax.experimental.pallas import tpu as pltpu
from jax.experimental.pallas import tpu_sc as plsc

VOCAB = 1000000
DIM = 64
BATCH = 16384
HIST = 50

_B = BATCH * HIST
_ROW = 128
_NW = 32
_ROWS_TOTAL = _B // _ROW
_ROWS_PER_W = _ROWS_TOTAL // _NW


def _gather_body(idx_hbm, table_hbm, out_hbm, idx_v, rows_v, gsem):
    wid = lax.axis_index("s") * 2 + lax.axis_index("c")
    row_base = wid * _ROWS_PER_W
    pltpu.sync_copy(idx_hbm.at[pl.ds(row_base, _ROWS_PER_W), :], idx_v)

    def step(j, carry):
        pltpu.async_copy(table_hbm.at[idx_v.at[j]], rows_v, gsem).wait()
        pltpu.sync_copy(
            rows_v, out_hbm.at[pl.ds((row_base + j) * _ROW, _ROW), :]
        )
        return carry

    lax.fori_loop(0, _ROWS_PER_W, step, 0)


@jax.jit
def kernel(x, table):
    idx = x.reshape(_ROWS_TOTAL, _ROW).astype(jnp.int32)
    mesh = plsc.VectorSubcoreMesh(core_axis_name="c", subcore_axis_name="s")
    out = pl.kernel(
        _gather_body,
        out_type=jax.ShapeDtypeStruct((_B, DIM), jnp.float32),
        mesh=mesh,
        scratch_types=[
            pltpu.VMEM((_ROWS_PER_W, _ROW), jnp.int32),
            pltpu.VMEM((_ROW, DIM), jnp.float32),
            pltpu.SemaphoreType.DMA,
        ],
        compiler_params=pltpu.CompilerParams(use_tc_tiling_on_sc=False),
    )(idx, table)
    return out.reshape(BATCH, HIST, DIM)


# speedup vs baseline: 1.8768x; 1.1066x over previous
import functools

import jax
import jax.numpy as jnp
from jax import lax
from jax.experimental import pallas as pl
from jax.experimental.pallas import tpu as pltpu
from jax.experimental.pallas import tpu_sc as plsc

VOCAB = 1000000
DIM = 64
BATCH = 16384
HIST = 50

_B = BATCH * HIST
_ROW = 128
_NW = 32
_ROWS_TOTAL = _B // _ROW
_ROWS_PER_W = _ROWS_TOTAL // _NW


_K = 4
_CHUNK = _K * _ROW
_M = _ROWS_PER_W // _K


def _gather_body(idx_hbm, table_hbm, out_hbm, idx_v, buf0, buf1, gsem0, gsem1):
    wid = lax.axis_index("s") * 2 + lax.axis_index("c")
    row_base = wid * _ROWS_PER_W
    pltpu.sync_copy(idx_hbm.at[pl.ds(row_base, _ROWS_PER_W), :], idx_v)

    bufs = (buf0, buf1)
    gsems = (gsem0, gsem1)

    def fire(c, b):
        for j in range(_K):
            pltpu.async_copy(
                table_hbm.at[idx_v.at[c * _K + j]],
                bufs[b].at[pl.ds(j * _ROW, _ROW), :],
                gsems[b],
            )

    def drain(b):
        for j in range(_K):
            pltpu.make_async_copy(
                table_hbm.at[idx_v.at[j]],
                bufs[b].at[pl.ds(j * _ROW, _ROW), :],
                gsems[b],
            ).wait()

    fire(0, 0)

    @pl.loop(0, _M, step=2)
    def _(c):
        for b in range(2):
            cc = c + b

            @pl.when(cc + 1 < _M)
            def _():
                fire(cc + 1, 1 - b)

            drain(b)
            pltpu.sync_copy(
                bufs[b],
                out_hbm.at[pl.ds((row_base + cc * _K) * _ROW, _CHUNK), :],
            )


@jax.jit
def kernel(x, table):
    idx = x.reshape(_ROWS_TOTAL, _ROW).astype(jnp.int32)
    mesh = plsc.VectorSubcoreMesh(core_axis_name="c", subcore_axis_name="s")
    out = pl.kernel(
        _gather_body,
        out_type=jax.ShapeDtypeStruct((_B, DIM), jnp.float32),
        mesh=mesh,
        scratch_types=[
            pltpu.VMEM((_ROWS_PER_W, _ROW), jnp.int32),
            pltpu.VMEM((_CHUNK, DIM), jnp.float32),
            pltpu.VMEM((_CHUNK, DIM), jnp.float32),
            pltpu.SemaphoreType.DMA,
            pltpu.SemaphoreType.DMA,
        ],
        compiler_params=pltpu.CompilerParams(use_tc_tiling_on_sc=False),
    )(idx, table)
    return out.reshape(BATCH, HIST, DIM)
